# baseline (speedup 1.0000x reference)
import jax
import jax.numpy as jnp
from jax import lax
from jax.experimental import pallas as pl
from jax.experimental.pallas import tpu as pltpu

_B = 2048
_NSUB = 2


def _rq_body(data_ref, cbt_ref, cb3_ref, c2_ref, codes_ref, recon_ref):
    n_layers, d, n_clusters = cbt_ref.shape
    sb = _B // _NSUB
    iota = lax.broadcasted_iota(jnp.int32, (sb, n_clusters), 1)
    rs = [data_ref[pl.ds(h * sb, sb), :] for h in range(_NSUB)]
    qsums = [jnp.zeros_like(rs[0]) for _ in range(_NSUB)]
    for l in range(n_layers):
        for h in range(_NSUB):
            dots = lax.dot_general(
                rs[h], cbt_ref[l], (((1,), (0,)), ((), ())),
                precision=lax.Precision.DEFAULT,
                preferred_element_type=jnp.float32)
            s = dots + c2_ref[l:l + 1, :]
            idx = jnp.argmin(s, axis=1).astype(jnp.int32)[:, None]
            codes_ref[pl.ds(h * sb, sb), l:l + 1] = idx
            onehot = (iota == idx).astype(jnp.bfloat16)
            q3 = lax.dot_general(
                onehot, cb3_ref[l], (((1,), (0,)), ((), ())),
                preferred_element_type=jnp.float32)
            q = (q3[:, :d] + q3[:, d:2 * d]) + q3[:, 2 * d:]
            rs[h] = rs[h] - q
            qsums[h] = qsums[h] + q
    for h in range(_NSUB):
        recon_ref[pl.ds(h * sb, sb), :] = qsums[h]


def kernel(data, codebooks):
    n, d = data.shape
    n_layers, n_clusters, _ = codebooks.shape
    cbt = jnp.swapaxes(codebooks, 1, 2) * -2.0
    c2 = jnp.sum(codebooks * codebooks, axis=2)

    def _top16(x):
        return jax.lax.bitcast_convert_type(
            jax.lax.bitcast_convert_type(x, jnp.uint32) & jnp.uint32(0xFFFF0000),
            jnp.float32)

    hi_f = _top16(codebooks)
    r1 = codebooks - hi_f
    mid_f = _top16(r1)
    r2 = r1 - mid_f
    cb3 = jnp.concatenate(
        [hi_f.astype(jnp.bfloat16),
         mid_f.astype(jnp.bfloat16),
         _top16(r2).astype(jnp.bfloat16)], axis=2)

    codes32, recon = pl.pallas_call(
        _rq_body,
        grid=(n // _B,),
        in_specs=[
            pl.BlockSpec((_B, d), lambda i: (i, 0)),
            pl.BlockSpec((n_layers, d, n_clusters), lambda i: (0, 0, 0)),
            pl.BlockSpec((n_layers, n_clusters, 3 * d), lambda i: (0, 0, 0)),
            pl.BlockSpec((n_layers, n_clusters), lambda i: (0, 0)),
        ],
        out_specs=[
            pl.BlockSpec((_B, n_layers), lambda i: (i, 0)),
            pl.BlockSpec((_B, d), lambda i: (i, 0)),
        ],
        out_shape=[
            jax.ShapeDtypeStruct((n, n_layers), jnp.int32),
            jax.ShapeDtypeStruct((n, d), jnp.float32),
        ],
        compiler_params=pltpu.CompilerParams(
            dimension_semantics=("parallel",)),
    )(data, cbt, cb3, c2)
    return codes32.astype(jnp.int64), recon


# speedup vs baseline: 1.1236x; 1.1236x over previous
import jax
import jax.numpy as jnp
from jax import lax
from jax.experimental import pallas as pl
from jax.experimental.pallas import tpu as pltpu

_B = 2048
_NSUB = 2


def _rq_body(data_ref, cbt_ref, cb3_ref, c2_ref, codes_ref, recon_ref):
    n_layers, d, n_clusters = cbt_ref.shape
    sb = _B // _NSUB
    iota = lax.broadcasted_iota(
        jnp.int32, (sb, n_clusters), 1).astype(jnp.float32)
    rs = [data_ref[pl.ds(h * sb, sb), :] for h in range(_NSUB)]
    qsums = [jnp.zeros_like(rs[0]) for _ in range(_NSUB)]
    for l in range(n_layers):
        for h in range(_NSUB):
            dots = lax.dot_general(
                rs[h], cbt_ref[l], (((1,), (0,)), ((), ())),
                precision=lax.Precision.DEFAULT,
                preferred_element_type=jnp.float32)
            s = dots + c2_ref[l:l + 1, :]
            m = jnp.min(s, axis=1, keepdims=True)
            idx = jnp.min(jnp.where(s == m, iota, jnp.float32(n_clusters)),
                          axis=1, keepdims=True)
            codes_ref[pl.ds(h * sb, sb), l:l + 1] = idx.astype(jnp.int32)
            onehot = (iota == idx).astype(jnp.bfloat16)
            q3 = lax.dot_general(
                onehot, cb3_ref[l], (((1,), (0,)), ((), ())),
                preferred_element_type=jnp.float32)
            q = (q3[:, :d] + q3[:, d:2 * d]) + q3[:, 2 * d:]
            rs[h] = rs[h] - q
            qsums[h] = qsums[h] + q
    for h in range(_NSUB):
        recon_ref[pl.ds(h * sb, sb), :] = qsums[h]


def kernel(data, codebooks):
    n, d = data.shape
    n_layers, n_clusters, _ = codebooks.shape
    cbt = jnp.swapaxes(codebooks, 1, 2) * -2.0
    c2 = jnp.sum(codebooks * codebooks, axis=2)

    def _top16(x):
        return jax.lax.bitcast_convert_type(
            jax.lax.bitcast_convert_type(x, jnp.uint32) & jnp.uint32(0xFFFF0000),
            jnp.float32)

    hi_f = _top16(codebooks)
    r1 = codebooks - hi_f
    mid_f = _top16(r1)
    r2 = r1 - mid_f
    cb3 = jnp.concatenate(
        [hi_f.astype(jnp.bfloat16),
         mid_f.astype(jnp.bfloat16),
         _top16(r2).astype(jnp.bfloat16)], axis=2)

    codes32, recon = pl.pallas_call(
        _rq_body,
        grid=(n // _B,),
        in_specs=[
            pl.BlockSpec((_B, d), lambda i: (i, 0)),
            pl.BlockSpec((n_layers, d, n_clusters), lambda i: (0, 0, 0)),
            pl.BlockSpec((n_layers, n_clusters, 3 * d), lambda i: (0, 0, 0)),
            pl.BlockSpec((n_layers, n_clusters), lambda i: (0, 0)),
        ],
        out_specs=[
            pl.BlockSpec((_B, n_layers), lambda i: (i, 0)),
            pl.BlockSpec((_B, d), lambda i: (i, 0)),
        ],
        out_shape=[
            jax.ShapeDtypeStruct((n, n_layers), jnp.int32),
            jax.ShapeDtypeStruct((n, d), jnp.float32),
        ],
        compiler_params=pltpu.CompilerParams(
            dimension_semantics=("parallel",)),
    )(data, cbt, cb3, c2)
    return codes32.astype(jnp.int64), recon
